# Initial kernel scaffold; baseline (speedup 1.0000x reference)
import jax
import jax.numpy as jnp
from jax.experimental import pallas as pl


def kernel(center_vecs, id2center, doc_ids, neg_ids):
    raise NotImplementedError("write your pallas kernel here")


# speedup vs baseline: 12.4284x; 12.4284x over previous
import functools

import jax
import jax.numpy as jnp
from jax import lax
from jax.experimental import pallas as pl
from jax.experimental.pallas import tpu as pltpu
from jax.experimental.pallas import tpu_sc as plsc

K_CENTERS = 65536
DIM = 128
BATCH = 4096

NUM_CORES = 2
NUM_SUBCORES = 16
NW = NUM_CORES * NUM_SUBCORES
CHUNK = 128


def _body(center_hbm, id2center_hbm, ids_hbm, out_hbm, idx_v, cid_v, rows_v, sem):
    wid = lax.axis_index("s") * NUM_CORES + lax.axis_index("c")
    b_per_w = ids_hbm.shape[0] // NW
    nchunks = b_per_w // CHUNK
    base = wid * b_per_w
    for j in range(nchunks):
        off = base + j * CHUNK
        pltpu.sync_copy(ids_hbm.at[pl.ds(off, CHUNK)], idx_v.at[j])
        pltpu.async_copy(id2center_hbm.at[idx_v.at[j]], cid_v.at[j], sem).wait()
        pltpu.async_copy(center_hbm.at[cid_v.at[j]], rows_v.at[j], sem).wait()
        pltpu.sync_copy(rows_v.at[j], out_hbm.at[pl.ds(off, CHUNK)])


@jax.jit
def _ivf_lookup(center_vecs, id2center, ids):
    b = ids.shape[0]
    nchunks = b // NW // CHUNK
    run = functools.partial(
        pl.kernel,
        out_type=jax.ShapeDtypeStruct((b, DIM), jnp.float32),
        mesh=plsc.VectorSubcoreMesh(core_axis_name="c", subcore_axis_name="s"),
        scratch_types=[
            pltpu.VMEM((nchunks, CHUNK), jnp.int32),
            pltpu.VMEM((nchunks, CHUNK), jnp.int32),
            pltpu.VMEM((nchunks, CHUNK, DIM), jnp.float32),
            pltpu.SemaphoreType.DMA,
        ],
    )(_body)
    return run(center_vecs, id2center, ids)


def kernel(center_vecs, id2center, doc_ids, neg_ids):
    ids = jnp.concatenate([doc_ids, neg_ids], axis=0)
    out = _ivf_lookup(center_vecs, id2center, ids)
    return out[:BATCH], out[BATCH:]


# speedup vs baseline: 13.3383x; 1.0732x over previous
import functools

import jax
import jax.numpy as jnp
from jax import lax
from jax.experimental import pallas as pl
from jax.experimental.pallas import tpu as pltpu
from jax.experimental.pallas import tpu_sc as plsc

K_CENTERS = 65536
DIM = 128
BATCH = 4096

NUM_CORES = 2
NUM_SUBCORES = 16
NW = NUM_CORES * NUM_SUBCORES
CHUNK = 128


def _body(center_hbm, id2center_hbm, ids_hbm, out_hbm, idx_v, cid_v, rows_v,
          *sems):
    wid = lax.axis_index("s") * NUM_CORES + lax.axis_index("c")
    b_per_w = ids_hbm.shape[0] // NW
    nchunks = b_per_w // CHUNK
    base = wid * b_per_w
    s_stage = sems[0:nchunks]
    s_cid = sems[nchunks:2 * nchunks]
    s_rows = sems[2 * nchunks:3 * nchunks]
    s_out = sems[3 * nchunks:4 * nchunks]

    stage = [
        pltpu.async_copy(ids_hbm.at[pl.ds(base + j * CHUNK, CHUNK)],
                         idx_v.at[j], s_stage[j])
        for j in range(nchunks)
    ]
    cid = []
    for j in range(nchunks):
        stage[j].wait()
        cid.append(pltpu.async_copy(id2center_hbm.at[idx_v.at[j]],
                                    cid_v.at[j], s_cid[j]))
    rows = []
    for j in range(nchunks):
        cid[j].wait()
        rows.append(pltpu.async_copy(center_hbm.at[cid_v.at[j]],
                                     rows_v.at[j], s_rows[j]))
    outs = []
    for j in range(nchunks):
        rows[j].wait()
        outs.append(pltpu.async_copy(rows_v.at[j],
                                     out_hbm.at[pl.ds(base + j * CHUNK, CHUNK)],
                                     s_out[j]))
    for j in range(nchunks):
        outs[j].wait()


@jax.jit
def _ivf_lookup(center_vecs, id2center, ids):
    b = ids.shape[0]
    nchunks = b // NW // CHUNK
    run = functools.partial(
        pl.kernel,
        out_type=jax.ShapeDtypeStruct((b, DIM), jnp.float32),
        mesh=plsc.VectorSubcoreMesh(core_axis_name="c", subcore_axis_name="s"),
        scratch_types=[
            pltpu.VMEM((nchunks, CHUNK), jnp.int32),
            pltpu.VMEM((nchunks, CHUNK), jnp.int32),
            pltpu.VMEM((nchunks, CHUNK, DIM), jnp.float32),
        ] + [pltpu.SemaphoreType.DMA] * (4 * nchunks),
    )(_body)
    return run(center_vecs, id2center, ids)


def kernel(center_vecs, id2center, doc_ids, neg_ids):
    ids = jnp.concatenate([doc_ids, neg_ids], axis=0)
    out = _ivf_lookup(center_vecs, id2center, ids)
    return out[:BATCH], out[BATCH:]


# speedup vs baseline: 15.7216x; 1.1787x over previous
import functools

import jax
import jax.numpy as jnp
from jax import lax
from jax.experimental import pallas as pl
from jax.experimental.pallas import tpu as pltpu
from jax.experimental.pallas import tpu_sc as plsc

DIM = 128
BATCH = 4096

NUM_CORES = 2
NUM_SUBCORES = 16
NW = NUM_CORES * NUM_SUBCORES
CHUNK = BATCH // NW


def _body(center_hbm, id2center_hbm, doc_hbm, neg_hbm, dc_hbm, nc_hbm,
          idx_v, cid_v, rows_v, *sems):
    wid = lax.axis_index("s") * NUM_CORES + lax.axis_index("c")
    base = wid * CHUNK
    ids_refs = (doc_hbm, neg_hbm)
    out_refs = (dc_hbm, nc_hbm)
    s_stage, s_cid, s_rows, s_out = (sems[0:2], sems[2:4], sems[4:6], sems[6:8])

    stage = [
        pltpu.async_copy(ids_refs[j].at[pl.ds(base, CHUNK)], idx_v.at[j],
                         s_stage[j])
        for j in range(2)
    ]
    cid = []
    for j in range(2):
        stage[j].wait()
        cid.append(pltpu.async_copy(id2center_hbm.at[idx_v.at[j]],
                                    cid_v.at[j], s_cid[j]))
    rows = []
    for j in range(2):
        cid[j].wait()
        rows.append(pltpu.async_copy(center_hbm.at[cid_v.at[j]],
                                     rows_v.at[j], s_rows[j]))
    outs = []
    for j in range(2):
        rows[j].wait()
        outs.append(pltpu.async_copy(rows_v.at[j],
                                     out_refs[j].at[pl.ds(base, CHUNK)],
                                     s_out[j]))
    for j in range(2):
        outs[j].wait()


@jax.jit
def _ivf_lookup(center_vecs, id2center, doc_ids, neg_ids):
    run = functools.partial(
        pl.kernel,
        out_type=(
            jax.ShapeDtypeStruct((BATCH, DIM), jnp.float32),
            jax.ShapeDtypeStruct((BATCH, DIM), jnp.float32),
        ),
        mesh=plsc.VectorSubcoreMesh(core_axis_name="c", subcore_axis_name="s"),
        scratch_types=[
            pltpu.VMEM((2, CHUNK), jnp.int32),
            pltpu.VMEM((2, CHUNK), jnp.int32),
            pltpu.VMEM((2, CHUNK, DIM), jnp.float32),
        ] + [pltpu.SemaphoreType.DMA] * 8,
    )(_body)
    return run(center_vecs, id2center, doc_ids, neg_ids)


def kernel(center_vecs, id2center, doc_ids, neg_ids):
    return _ivf_lookup(center_vecs, id2center, doc_ids, neg_ids)
